# baseline (speedup 1.0000x reference)
import functools

import jax
import jax.numpy as jnp
from jax import lax
from jax.experimental import pallas as pl
from jax.experimental.pallas import tpu as pltpu
from jax.experimental.pallas import tpu_sc as plsc

IMAGE_TOKEN_ID = 151655


@functools.lru_cache(maxsize=None)
def _build_sc_kernel(seq_len: int, hidden: int, num_img: int):
    info = plsc.get_sparse_core_info()
    nc, ns, lanes = info.num_cores, info.num_subcores, info.num_lanes
    nw = nc * ns
    assert seq_len % nw == 0
    chunk = seq_len // nw
    assert chunk == lanes
    nchunks = seq_len // lanes

    mesh = plsc.VectorSubcoreMesh(core_axis_name="c", subcore_axis_name="s")

    @functools.partial(
        pl.kernel,
        out_type=jax.ShapeDtypeStruct((seq_len + nw, hidden), jnp.float32),
        mesh=mesh,
        compiler_params=pltpu.CompilerParams(needs_layout_passes=False),
        scratch_types=[
            pltpu.VMEM((seq_len,), jnp.int32),
            pltpu.VMEM((lanes,), jnp.int32),
            pltpu.VMEM((lanes,), jnp.int32),
            pltpu.VMEM((lanes,), jnp.int32),
            pltpu.VMEM((lanes,), jnp.int32),
            pltpu.VMEM((lanes, hidden), jnp.float32),
            pltpu.VMEM((lanes, hidden), jnp.float32),
            pltpu.SemaphoreType.DMA,
            pltpu.SemaphoreType.DMA,
        ],
    )
    def sc_kernel(emb_hbm, img_hbm, ids_hbm, out_hbm,
                  ids_v, idx_text_v, idx_img_v, pos_text_v, pos_img_v,
                  text_v, img_v, sem_a, sem_b):
        wid = lax.axis_index("s") * nc + lax.axis_index("c")
        base = wid * chunk

        pltpu.sync_copy(ids_hbm, ids_v)

        nbefore = jnp.zeros((lanes,), jnp.int32)
        my_ids = jnp.zeros((lanes,), jnp.int32)
        for j in range(nchunks):
            v = ids_v[pl.ds(j * lanes, lanes)]
            cnt = plsc.all_reduce_population_count(v == IMAGE_TOKEN_ID)
            nbefore = nbefore + jnp.where(j < wid, cnt, 0)
            my_ids = jnp.where(jnp.int32(j) == wid, v, my_ids)

        mask = my_ids == IMAGE_TOKEN_ID
        n_img = jnp.max(plsc.all_reduce_population_count(mask))
        intra = plsc.cumsum(mask.astype(jnp.int32))
        ordinal = jnp.clip(nbefore + intra - 1, 0, num_img - 1)
        pos = base + lax.iota(jnp.int32, lanes)
        dummy = seq_len + wid

        half = lanes // 2

        def piped_copy(src_hbm, idx_ref, buf):
            ga = pltpu.async_copy(
                src_hbm.at[idx_ref.at[pl.ds(0, half)]], buf.at[pl.ds(0, half)], sem_a)
            ga.wait()
            sa = pltpu.async_copy(
                buf.at[pl.ds(0, half)], out_hbm.at[pl.ds(base, half)], sem_b)
            gb = pltpu.async_copy(
                src_hbm.at[idx_ref.at[pl.ds(half, half)]], buf.at[pl.ds(half, half)], sem_a)
            gb.wait()
            sb = pltpu.async_copy(
                buf.at[pl.ds(half, half)], out_hbm.at[pl.ds(base + half, half)], sem_b)
            sa.wait()
            sb.wait()

        @pl.when(n_img == 0)
        def _():
            idx_text_v[...] = my_ids
            piped_copy(emb_hbm, idx_text_v, text_v)

        @pl.when(n_img == lanes)
        def _():
            idx_img_v[...] = ordinal
            piped_copy(img_hbm, idx_img_v, img_v)

        @pl.when(jnp.logical_and(n_img > 0, n_img < lanes))
        def _():
            idx_text_v[...] = jnp.where(mask, 0, my_ids)
            idx_img_v[...] = jnp.where(mask, ordinal, 0)
            pos_text_v[...] = jnp.where(mask, dummy, pos)
            pos_img_v[...] = jnp.where(mask, pos, dummy)
            cp_t = pltpu.async_copy(emb_hbm.at[idx_text_v], text_v, sem_a)
            cp_i = pltpu.async_copy(img_hbm.at[idx_img_v], img_v, sem_b)
            cp_t.wait()
            st_t = pltpu.async_copy(text_v, out_hbm.at[pos_text_v], sem_a)
            cp_i.wait()
            st_i = pltpu.async_copy(img_v, out_hbm.at[pos_img_v], sem_b)
            st_t.wait()
            st_i.wait()

    return sc_kernel


def kernel(input_ids, image_embeds, embed_weight):
    batch, seq_len = input_ids.shape
    num_img, hidden = image_embeds.shape
    ids = input_ids.reshape(seq_len).astype(jnp.int32)
    sc = _build_sc_kernel(seq_len, hidden, num_img)
    out_pad = sc(embed_weight, image_embeds, ids)
    return out_pad[:seq_len].reshape(batch, seq_len, hidden)


# speedup vs baseline: 1.2078x; 1.2078x over previous
import functools

import jax
import jax.numpy as jnp
from jax import lax
from jax.experimental import pallas as pl
from jax.experimental.pallas import tpu as pltpu
from jax.experimental.pallas import tpu_sc as plsc

IMAGE_TOKEN_ID = 151655


@functools.lru_cache(maxsize=None)
def _build_sc_kernel(seq_len: int, hidden: int, num_img: int):
    info = plsc.get_sparse_core_info()
    nc, ns, lanes = info.num_cores, info.num_subcores, info.num_lanes
    nw = nc * ns
    assert seq_len % nw == 0
    chunk = seq_len // nw
    assert chunk == lanes
    nchunks = seq_len // lanes

    mesh = plsc.VectorSubcoreMesh(core_axis_name="c", subcore_axis_name="s")

    @functools.partial(
        pl.kernel,
        out_type=jax.ShapeDtypeStruct((seq_len, hidden), jnp.float32),
        mesh=mesh,
        compiler_params=pltpu.CompilerParams(needs_layout_passes=False),
        scratch_types=[
            pltpu.VMEM((seq_len,), jnp.int32),
            pltpu.VMEM((lanes,), jnp.int32),
            pltpu.VMEM((lanes,), jnp.int32),
            pltpu.VMEM((lanes,), jnp.int32),
            pltpu.VMEM((lanes,), jnp.int32),
            pltpu.VMEM((lanes, hidden), jnp.float32),
            pltpu.VMEM((lanes, hidden), jnp.float32),
            pltpu.SemaphoreType.DMA,
            pltpu.SemaphoreType.DMA,
        ],
    )
    def sc_kernel(emb_hbm, img_hbm, ids_hbm, out_hbm,
                  ids_v, idx_text_v, idx_img_v, pos_text_v, pos_img_v,
                  text_v, img_v, sem_a, sem_b):
        wid = lax.axis_index("c") * ns + lax.axis_index("s")
        base = wid * chunk

        pltpu.sync_copy(ids_hbm, ids_v)

        nbefore = jnp.zeros((lanes,), jnp.int32)
        my_ids = jnp.zeros((lanes,), jnp.int32)
        for j in range(nchunks):
            v = ids_v[pl.ds(j * lanes, lanes)]
            cnt = plsc.all_reduce_population_count(v == IMAGE_TOKEN_ID)
            nbefore = nbefore + jnp.where(j < wid, cnt, 0)
            my_ids = jnp.where(jnp.int32(j) == wid, v, my_ids)

        mask = my_ids == IMAGE_TOKEN_ID
        n_img = jnp.max(plsc.all_reduce_population_count(mask))
        intra = plsc.cumsum(mask.astype(jnp.int32))
        ordinal = jnp.clip(nbefore + intra - 1, 0, num_img - 1)
        lane = lax.iota(jnp.int32, lanes)
        pos = base + lane

        half = lanes // 2

        def piped_copy(src_hbm, idx_ref, buf):
            ga = pltpu.async_copy(
                src_hbm.at[idx_ref.at[pl.ds(0, half)]], buf.at[pl.ds(0, half)], sem_a)
            ga.wait()
            sa = pltpu.async_copy(
                buf.at[pl.ds(0, half)], out_hbm.at[pl.ds(base, half)], sem_b)
            gb = pltpu.async_copy(
                src_hbm.at[idx_ref.at[pl.ds(half, half)]], buf.at[pl.ds(half, half)], sem_a)
            gb.wait()
            sb = pltpu.async_copy(
                buf.at[pl.ds(half, half)], out_hbm.at[pl.ds(base + half, half)], sem_b)
            sa.wait()
            sb.wait()

        @pl.when(n_img == 0)
        def _():
            idx_text_v[...] = my_ids
            piped_copy(emb_hbm, idx_text_v, text_v)

        @pl.when(n_img == lanes)
        def _():
            idx_img_v[...] = ordinal
            piped_copy(img_hbm, idx_img_v, img_v)

        @pl.when(jnp.logical_and(n_img > 0, n_img < lanes))
        def _():
            ft = jnp.min(jnp.where(mask, lanes, lane))
            fi = jnp.min(jnp.where(mask, lane, lanes))
            ftv = jnp.zeros((lanes,), jnp.int32) + ft
            first_text_id = plsc.load_gather(ids_v, [base + ftv])
            idx_text_v[...] = jnp.where(mask, first_text_id, my_ids)
            idx_img_v[...] = jnp.where(mask, ordinal, jnp.clip(nbefore, 0, num_img - 1))
            pos_text_v[...] = jnp.where(mask, base + ft, pos)
            pos_img_v[...] = jnp.where(mask, pos, base + fi)
            cp_t = pltpu.async_copy(emb_hbm.at[idx_text_v], text_v, sem_a)
            cp_i = pltpu.async_copy(img_hbm.at[idx_img_v], img_v, sem_b)
            cp_t.wait()
            st_t = pltpu.async_copy(text_v, out_hbm.at[pos_text_v], sem_a)
            cp_i.wait()
            st_i = pltpu.async_copy(img_v, out_hbm.at[pos_img_v], sem_b)
            st_t.wait()
            st_i.wait()

    return sc_kernel


def kernel(input_ids, image_embeds, embed_weight):
    batch, seq_len = input_ids.shape
    num_img, hidden = image_embeds.shape
    ids = input_ids.reshape(seq_len).astype(jnp.int32)
    sc = _build_sc_kernel(seq_len, hidden, num_img)
    out = sc(embed_weight, image_embeds, ids)
    return out.reshape(batch, seq_len, hidden)


# speedup vs baseline: 1.2084x; 1.0004x over previous
import functools

import jax
import jax.numpy as jnp
from jax import lax
from jax.experimental import pallas as pl
from jax.experimental.pallas import tpu as pltpu
from jax.experimental.pallas import tpu_sc as plsc

IMAGE_TOKEN_ID = 151655


@functools.lru_cache(maxsize=None)
def _build_sc_kernel(seq_len: int, hidden: int, num_img: int):
    info = plsc.get_sparse_core_info()
    nc, ns, lanes = info.num_cores, info.num_subcores, info.num_lanes
    nw = nc * ns
    assert seq_len % nw == 0
    chunk = seq_len // nw
    assert chunk == lanes
    nchunks = seq_len // lanes

    mesh = plsc.VectorSubcoreMesh(core_axis_name="c", subcore_axis_name="s")

    @functools.partial(
        pl.kernel,
        out_type=jax.ShapeDtypeStruct((seq_len, hidden), jnp.float32),
        mesh=mesh,
        compiler_params=pltpu.CompilerParams(
            needs_layout_passes=False,
            skip_device_barrier=True,
            disable_bounds_checks=True,
            disable_semaphore_checks=True,
        ),
        scratch_types=[
            pltpu.VMEM((seq_len,), jnp.int32),
            pltpu.VMEM((lanes,), jnp.int32),
            pltpu.VMEM((lanes,), jnp.int32),
            pltpu.VMEM((lanes,), jnp.int32),
            pltpu.VMEM((lanes,), jnp.int32),
            pltpu.VMEM((lanes, hidden), jnp.float32),
            pltpu.VMEM((lanes, hidden), jnp.float32),
            pltpu.SemaphoreType.DMA,
            pltpu.SemaphoreType.DMA,
        ],
    )
    def sc_kernel(emb_hbm, img_hbm, ids_hbm, out_hbm,
                  ids_v, idx_text_v, idx_img_v, pos_text_v, pos_img_v,
                  text_v, img_v, sem_a, sem_b):
        wid = lax.axis_index("c") * ns + lax.axis_index("s")
        base = wid * chunk

        pltpu.sync_copy(ids_hbm, ids_v)

        nbefore = jnp.zeros((lanes,), jnp.int32)
        my_ids = jnp.zeros((lanes,), jnp.int32)
        for j in range(nchunks):
            v = ids_v[pl.ds(j * lanes, lanes)]
            cnt = plsc.all_reduce_population_count(v == IMAGE_TOKEN_ID)
            nbefore = nbefore + jnp.where(j < wid, cnt, 0)
            my_ids = jnp.where(jnp.int32(j) == wid, v, my_ids)

        mask = my_ids == IMAGE_TOKEN_ID
        n_img = jnp.max(plsc.all_reduce_population_count(mask))
        intra = plsc.cumsum(mask.astype(jnp.int32))
        ordinal = jnp.clip(nbefore + intra - 1, 0, num_img - 1)
        lane = lax.iota(jnp.int32, lanes)
        pos = base + lane

        half = lanes // 2

        def piped_copy(src_hbm, idx_ref, buf):
            ga = pltpu.async_copy(
                src_hbm.at[idx_ref.at[pl.ds(0, half)]], buf.at[pl.ds(0, half)], sem_a)
            ga.wait()
            sa = pltpu.async_copy(
                buf.at[pl.ds(0, half)], out_hbm.at[pl.ds(base, half)], sem_b)
            gb = pltpu.async_copy(
                src_hbm.at[idx_ref.at[pl.ds(half, half)]], buf.at[pl.ds(half, half)], sem_a)
            gb.wait()
            sb = pltpu.async_copy(
                buf.at[pl.ds(half, half)], out_hbm.at[pl.ds(base + half, half)], sem_b)
            sa.wait()
            sb.wait()

        @pl.when(n_img == 0)
        def _():
            idx_text_v[...] = my_ids
            piped_copy(emb_hbm, idx_text_v, text_v)

        @pl.when(n_img == lanes)
        def _():
            idx_img_v[...] = ordinal
            piped_copy(img_hbm, idx_img_v, img_v)

        @pl.when(jnp.logical_and(n_img > 0, n_img < lanes))
        def _():
            ft = jnp.min(jnp.where(mask, lanes, lane))
            fi = jnp.min(jnp.where(mask, lane, lanes))
            ftv = jnp.zeros((lanes,), jnp.int32) + ft
            first_text_id = plsc.load_gather(ids_v, [base + ftv])
            idx_text_v[...] = jnp.where(mask, first_text_id, my_ids)
            idx_img_v[...] = jnp.where(mask, ordinal, jnp.clip(nbefore, 0, num_img - 1))
            pos_text_v[...] = jnp.where(mask, base + ft, pos)
            pos_img_v[...] = jnp.where(mask, pos, base + fi)
            cp_t = pltpu.async_copy(emb_hbm.at[idx_text_v], text_v, sem_a)
            cp_i = pltpu.async_copy(img_hbm.at[idx_img_v], img_v, sem_b)
            cp_t.wait()
            st_t = pltpu.async_copy(text_v, out_hbm.at[pos_text_v], sem_a)
            cp_i.wait()
            st_i = pltpu.async_copy(img_v, out_hbm.at[pos_img_v], sem_b)
            st_t.wait()
            st_i.wait()

    return sc_kernel


def kernel(input_ids, image_embeds, embed_weight):
    batch, seq_len = input_ids.shape
    num_img, hidden = image_embeds.shape
    ids = input_ids.reshape(seq_len).astype(jnp.int32)
    sc = _build_sc_kernel(seq_len, hidden, num_img)
    out = sc(embed_weight, image_embeds, ids)
    return out.reshape(batch, seq_len, hidden)
